# Initial kernel scaffold; baseline (speedup 1.0000x reference)
import jax
import jax.numpy as jnp
from jax.experimental import pallas as pl


def kernel(z, embed_w):
    raise NotImplementedError("write your pallas kernel here")


# speedup vs baseline: 9.3742x; 9.3742x over previous
import functools

import jax
import jax.numpy as jnp
from jax import lax
from jax.experimental import pallas as pl
from jax.experimental.pallas import tpu as pltpu
from jax.experimental.pallas import tpu_sc as plsc

_GROUPS = 4
_N_EMBED = 1024
_DSUB = 64


_TBLK = 512


def _score_kernel(z_ref, emb_ref, ind_ref):
    zb = z_ref[...]
    for g in range(_GROUPS):
        a = zb[:, g * _DSUB:(g + 1) * _DSUB]
        w = emb_ref[g * _N_EMBED:(g + 1) * _N_EMBED, :]
        s = lax.dot_general(
            a, w, dimension_numbers=(((1,), (1,)), ((), ())),
            preferred_element_type=jnp.float32,
        )
        f2 = jnp.sum(a * a, axis=1, keepdims=True)
        w2 = jnp.sum(w * w, axis=1)[None, :]
        dist = (f2 - 2.0 * s) + w2
        m = jnp.min(dist, axis=1, keepdims=True)
        iota = lax.broadcasted_iota(jnp.int32, dist.shape, 1)
        idx = jnp.min(jnp.where(dist == m, iota, _N_EMBED), axis=1)
        ind_ref[g, :] = idx.astype(jnp.int32)


def _compute_indices(z2, embed_w):
    nt = z2.shape[0]
    nb = nt // _TBLK
    return pl.pallas_call(
        _score_kernel,
        grid=(nb,),
        in_specs=[
            pl.BlockSpec((_TBLK, _GROUPS * _DSUB), lambda i: (i, 0)),
            pl.BlockSpec((_GROUPS * _N_EMBED, _DSUB), lambda i: (0, 0)),
        ],
        out_specs=pl.BlockSpec((_GROUPS, _TBLK), lambda i: (0, i)),
        out_shape=jax.ShapeDtypeStruct((_GROUPS, nt), jnp.int32),
    )(z2, embed_w)


_NC, _NS = 2, 16
_NW = _NC * _NS
_CHUNK = 128


def _make_gather(total_rows, d_model):
    rows_per_w = total_rows // _NW
    nchunk = rows_per_w // _CHUNK
    zrows = _CHUNK * _DSUB // d_model
    per_z = d_model // _DSUB
    mesh = plsc.VectorSubcoreMesh(core_axis_name="c", subcore_axis_name="s")

    @functools.partial(
        pl.kernel,
        mesh=mesh,
        out_type=[
            jax.ShapeDtypeStruct((total_rows // per_z, d_model), jnp.float32),
            jax.ShapeDtypeStruct((_NW, 128), jnp.float32),
        ],
        scratch_types=[
            pltpu.VMEM((nchunk, _CHUNK), jnp.int32),
            pltpu.VMEM((_CHUNK, 128), jnp.float32),
            pltpu.VMEM((zrows, d_model), jnp.float32),
            pltpu.VMEM((zrows, d_model), jnp.float32),
            pltpu.VMEM((128,), jnp.float32),
            pltpu.SemaphoreType.DMA,
        ],
    )
    def k(emb_hbm, idx_hbm, z_hbm, zq_hbm, psum_hbm, idx_v, rows_v, z_v,
          zqc_v, acc_v, sem):
        wid = lax.axis_index("s") * _NC + lax.axis_index("c")
        zbase = wid * (rows_per_w // per_z)
        pltpu.sync_copy(idx_hbm.at[wid], idx_v)
        acc = jnp.zeros((16,), jnp.float32)
        for j in range(nchunk):
            zrow0 = zbase + j * zrows
            gat = pltpu.async_copy(emb_hbm.at[idx_v.at[j]], rows_v, sem)
            pltpu.sync_copy(z_hbm.at[pl.ds(zrow0, zrows)], z_v)
            gat.wait()

            def body(zr, a):
                for q in range(d_model // 16):
                    fr = zr * per_z + q // (_DSUB // 16)
                    c = q % (_DSUB // 16)
                    g = rows_v[fr, pl.ds(c * 16, 16)]
                    d = g - z_v[zr, pl.ds(q * 16, 16)]
                    zqc_v[zr, pl.ds(q * 16, 16)] = g
                    a = a + d * d
                return a

            acc = lax.fori_loop(0, zrows, body, acc)
            pltpu.sync_copy(zqc_v, zq_hbm.at[pl.ds(zrow0, zrows)])
        for t in range(8):
            acc_v[pl.ds(t * 16, 16)] = jnp.zeros((16,), jnp.float32)
        acc_v[pl.ds(0, 16)] = acc
        pltpu.sync_copy(acc_v, psum_hbm.at[wid])

    return k


def kernel(z, embed_w):
    B, N, D = z.shape
    nt = B * N
    total = nt * _GROUPS
    z2 = z.reshape(nt, D)

    ind = _compute_indices(z2, embed_w)
    ind_flat = ind.reshape(total)

    idx3 = ind_flat.reshape(_NW, total // _NW // _CHUNK, _CHUNK)
    emb_pad = jnp.pad(embed_w[:_N_EMBED], ((0, 0), (0, 128 - _DSUB)))
    zq2, psum = _make_gather(total, D)(emb_pad, idx3, z2)

    diff = (12.5 / (total * _DSUB)) * jnp.sum(psum[:, :16])
    z_q = zq2.reshape(B, N, D)
    ind_out = ind_flat.reshape(N, B, _GROUPS)
    return (z_q, diff, ind_out)
